# Initial kernel scaffold; baseline (speedup 1.0000x reference)
import functools

import jax
import jax.numpy as jnp
from jax import lax
from jax.experimental import pallas as pl
from jax.experimental.pallas import tpu as pltpu
from jax.experimental.pallas import tpu_sc as plsc

NC = 2
NS = 16
NW = NC * NS
CHUNK = 128


def _sc_aggregate(x, src3, dst3, zeros, n_chunks):
    n_nodes, d = x.shape
    acc_rows = zeros.shape[0]
    zrows = acc_rows // NS
    orows = acc_rows // NS

    mesh = plsc.VectorSubcoreMesh(core_axis_name="c", subcore_axis_name="s")

    @functools.partial(
        pl.kernel,
        out_type=jax.ShapeDtypeStruct((NC, acc_rows, d), jnp.float32),
        mesh=mesh,
        scratch_types=[
            pltpu.VMEM((n_chunks, CHUNK), jnp.int32),
            pltpu.VMEM((n_chunks, CHUNK), jnp.int32),
            pltpu.VMEM((CHUNK, d), jnp.float32),
            pltpu.VMEM_SHARED((acc_rows, d), jnp.float32),
            pltpu.SemaphoreType.DMA,
        ],
    )
    def sc_kernel(x_hbm, src_hbm, dst_hbm, zeros_hbm, out_hbm,
                  src_v, dst_v, rows_v, acc, sem):
        c = lax.axis_index("c")
        s = lax.axis_index("s")
        wid = s * NC + c
        pltpu.sync_copy(zeros_hbm.at[pl.ds(s * zrows, zrows)],
                        acc.at[pl.ds(s * zrows, zrows)])
        pltpu.sync_copy(src_hbm.at[wid], src_v)
        pltpu.sync_copy(dst_hbm.at[wid], dst_v)
        plsc.subcore_barrier()

        def body(j, carry):
            pltpu.async_copy(x_hbm.at[src_v.at[j]], rows_v, sem).wait()
            pltpu.sync_copy(rows_v, acc.at[dst_v.at[j]], add=True)
            return carry

        lax.fori_loop(0, n_chunks, body, 0, unroll=False)
        plsc.subcore_barrier()
        pltpu.sync_copy(acc.at[pl.ds(s * orows, orows)],
                        out_hbm.at[c, pl.ds(s * orows, orows)])

    return sc_kernel(x, src3, dst3, zeros)


def _tc_combine_matmul(partials, W, n_nodes):
    d = W.shape[0]
    blk = 2000

    def body(p_ref, w_ref, o_ref):
        p = p_ref[...]
        ps = p[0] + p[1]
        o_ref[...] = lax.dot_general(
            ps, w_ref[...], (((1,), (1,)), ((), ())),
            preferred_element_type=jnp.float32)

    return pl.pallas_call(
        body,
        grid=(n_nodes // blk,),
        in_specs=[
            pl.BlockSpec((NC, blk, d), lambda i: (0, i, 0)),
            pl.BlockSpec((d, d), lambda i: (0, 0)),
        ],
        out_specs=pl.BlockSpec((blk, d), lambda i: (i, 0)),
        out_shape=jax.ShapeDtypeStruct((n_nodes, d), jnp.float32),
    )(partials[:, :n_nodes], W)


def kernel(x, edge_index, W):
    n_nodes, d = x.shape
    e = edge_index.shape[1]
    src = edge_index[0].astype(jnp.int32)
    dst = edge_index[1].astype(jnp.int32)

    n_chunks = -(-e // (NW * CHUNK))
    e_pad = NW * n_chunks * CHUNK
    if e_pad != e:
        src = jnp.concatenate([src, jnp.zeros((e_pad - e,), jnp.int32)])
        dst = jnp.concatenate([dst, jnp.full((e_pad - e,), n_nodes, jnp.int32)])
    src3 = src.reshape(NW, n_chunks, CHUNK)
    dst3 = dst.reshape(NW, n_chunks, CHUNK)

    acc_rows = -(-(n_nodes + 1) // NS) * NS
    zeros = jnp.zeros((acc_rows, d), jnp.float32)

    partials = _sc_aggregate(x, src3, dst3, zeros, n_chunks)
    return _tc_combine_matmul(partials, W, n_nodes)


# speedup vs baseline: 4.8988x; 4.8988x over previous
import functools

import jax
import jax.numpy as jnp
from jax import lax
from jax.experimental import pallas as pl
from jax.experimental.pallas import tpu as pltpu
from jax.experimental.pallas import tpu_sc as plsc

NC = 2
NS = 16
NW = NC * NS
CHUNK = 128


def _sc_aggregate(x, src3, dst3, zeros, n_chunks):
    n_nodes, d = x.shape
    acc_rows = zeros.shape[0]
    zrows = acc_rows // NS
    orows = acc_rows // NS

    mesh = plsc.VectorSubcoreMesh(core_axis_name="c", subcore_axis_name="s")

    @functools.partial(
        pl.kernel,
        out_type=jax.ShapeDtypeStruct((NC, acc_rows, d), jnp.float32),
        mesh=mesh,
        scratch_types=[
            pltpu.VMEM((n_chunks, CHUNK), jnp.int32),
            pltpu.VMEM((n_chunks, CHUNK), jnp.int32),
            pltpu.VMEM((CHUNK, d), jnp.float32),
            pltpu.VMEM_SHARED((acc_rows, d), jnp.float32),
            pltpu.SemaphoreType.DMA,
        ],
    )
    def sc_kernel(x_hbm, src_hbm, dst_hbm, zeros_hbm, out_hbm,
                  src_v, dst_v, rows_v, acc, sem):
        c = lax.axis_index("c")
        s = lax.axis_index("s")
        wid = s * NC + c
        pltpu.sync_copy(zeros_hbm.at[pl.ds(s * zrows, zrows)],
                        acc.at[pl.ds(s * zrows, zrows)])
        pltpu.sync_copy(src_hbm.at[wid], src_v)
        pltpu.sync_copy(dst_hbm.at[wid], dst_v)
        plsc.subcore_barrier()

        def body(j, carry):
            pltpu.async_copy(x_hbm.at[src_v.at[j]], rows_v, sem).wait()
            pltpu.sync_copy(rows_v, acc.at[dst_v.at[j]], add=True)
            return carry

        lax.fori_loop(0, n_chunks, body, 0, unroll=False)
        plsc.subcore_barrier()
        pltpu.sync_copy(acc.at[pl.ds(s * orows, orows)],
                        out_hbm.at[c, pl.ds(s * orows, orows)])

    return sc_kernel(x, src3, dst3, zeros)


def _tc_combine_matmul(partials, W, n_nodes):
    d = W.shape[0]
    blk = 2000

    def body(p_ref, w_ref, o_ref):
        p = p_ref[...]
        ps = p[0] + p[1]
        o_ref[...] = lax.dot_general(
            ps, w_ref[...], (((1,), (1,)), ((), ())),
            preferred_element_type=jnp.float32,
            precision=lax.Precision.HIGHEST)

    return pl.pallas_call(
        body,
        grid=(n_nodes // blk,),
        in_specs=[
            pl.BlockSpec((NC, blk, d), lambda i: (0, i, 0)),
            pl.BlockSpec((d, d), lambda i: (0, 0)),
        ],
        out_specs=pl.BlockSpec((blk, d), lambda i: (i, 0)),
        out_shape=jax.ShapeDtypeStruct((n_nodes, d), jnp.float32),
    )(partials[:, :n_nodes], W)


def kernel(x, edge_index, W):
    n_nodes, d = x.shape
    e = edge_index.shape[1]
    src = edge_index[0].astype(jnp.int32)
    dst = edge_index[1].astype(jnp.int32)

    n_chunks = -(-e // (NW * CHUNK))
    e_pad = NW * n_chunks * CHUNK
    if e_pad != e:
        src = jnp.concatenate([src, jnp.zeros((e_pad - e,), jnp.int32)])
        dst = jnp.concatenate([dst, jnp.full((e_pad - e,), n_nodes, jnp.int32)])
    src3 = src.reshape(NW, n_chunks, CHUNK)
    dst3 = dst.reshape(NW, n_chunks, CHUNK)

    acc_rows = -(-(n_nodes + 1) // (NS * 8)) * (NS * 8)
    zeros = jnp.zeros((acc_rows, d), jnp.float32)

    partials = _sc_aggregate(x, src3, dst3, zeros, n_chunks)
    return _tc_combine_matmul(partials, W, n_nodes)
